# baseline (speedup 1.0000x reference)
import functools

import jax
import jax.numpy as jnp
from jax import lax
from jax.experimental import pallas as pl
from jax.experimental.pallas import tpu as pltpu
from jax.experimental.pallas import tpu_sc as plsc

NC = 2
NS = 16
NW = NC * NS
LANES = 16


def _sc_kernel(B, L, W_words, V_pad, HALF):
    rows_per_tile = B // NW
    elems = rows_per_tile * L
    n_blocks = rows_per_tile // 16

    mesh = plsc.VectorSubcoreMesh(
        core_axis_name="c", subcore_axis_name="s",
        num_cores=NC, num_subcores=NS)

    @functools.partial(
        pl.kernel,
        out_type=jax.ShapeDtypeStruct((B,), jnp.float32),
        mesh=mesh,
        scratch_types=[
            pltpu.VMEM((W_words,), jnp.int32),
            pltpu.VMEM((V_pad,), jnp.float32),
            pltpu.VMEM((elems,), jnp.int32),
            pltpu.VMEM((rows_per_tile,), jnp.float32),
            pltpu.VMEM((LANES,), jnp.float32),
            pltpu.SemaphoreType.DMA,
        ],
        compiler_params=pltpu.CompilerParams(needs_layout_passes=False),
    )
    def body(g2l_hbm, emb_hbm, rules_hbm, bias_hbm, out_hbm,
             g2l_v, emb_v, rules_v, out_v, bias_v, sem):
        wid = lax.axis_index("s") * NC + lax.axis_index("c")
        base = wid * elems

        c1 = pltpu.async_copy(g2l_hbm, g2l_v, sem)
        c2 = pltpu.async_copy(emb_hbm, emb_v, sem)
        c3 = pltpu.async_copy(rules_hbm.at[pl.ds(base, elems)], rules_v, sem)
        c4 = pltpu.async_copy(bias_hbm, bias_v, sem)
        c1.wait()
        c2.wait()
        c3.wait()
        c4.wait()

        lane = lax.iota(jnp.int32, LANES)
        m_lo = lane < 8
        bias_vec = bias_v[...]

        def sq16(r):
            in_hi = r >= HALF
            word_idx = jnp.where(in_hi, r - HALF, r)
            w = plsc.load_gather(g2l_v, [word_idx])
            hi = jnp.bitwise_and(jnp.right_shift(w, 16), 0xFFFF)
            lo = jnp.bitwise_and(w, 0xFFFF)
            local = jnp.where(in_hi, hi, lo)
            v = plsc.load_gather(emb_v, [local])
            return v * v

        def blk(i, carry):
            blk_base = i * (16 * L)
            acc = jnp.zeros((LANES,), jnp.float32)
            for g in range(8):
                off = blk_base + g * (2 * L)
                s = jnp.zeros((LANES,), jnp.float32)
                t = jnp.zeros((LANES,), jnp.float32)
                for j in range(25):
                    sq = sq16(rules_v[pl.ds(off + j * LANES, LANES)])
                    if j < 12:
                        s = s + sq
                    elif j == 12:
                        s = s + jnp.where(m_lo, sq, 0.0)
                        t = t + jnp.where(m_lo, 0.0, sq)
                    else:
                        t = t + sq
                acc = jnp.where(lane == 2 * g, jnp.sum(s), acc)
                acc = jnp.where(lane == 2 * g + 1, jnp.sum(t), acc)
            out_v[pl.ds(i * 16, 16)] = acc + bias_vec
            return carry

        lax.fori_loop(0, n_blocks, blk, 0)
        pltpu.sync_copy(out_v, out_hbm.at[pl.ds(wid * rows_per_tile, rows_per_tile)])

    return body


def kernel(rules, global_to_local, emb_weight, bias):
    B, L = rules.shape
    V = emb_weight.shape[0]
    G = global_to_local.shape[0]

    gp = global_to_local.astype(jnp.int32)
    half = (G + 1) // 2
    packed = jnp.bitwise_or(gp[:half], jnp.left_shift(gp[half:2 * half], 16))
    W_words = (half + 15) // 16 * 16
    packed = jnp.pad(packed, (0, W_words - half))

    V_pad = (V + 15) // 16 * 16
    emb_p = jnp.pad(emb_weight.reshape(-1), (0, V_pad - V))

    bias_vec = jnp.broadcast_to(bias.reshape(()), (LANES,)).astype(jnp.float32)
    rules_flat = rules.reshape(-1).astype(jnp.int32)

    out = _sc_kernel(B, L, W_words, V_pad, half)(packed, emb_p, rules_flat, bias_vec)
    return out.reshape(B, 1)


# speedup vs baseline: 1.0860x; 1.0860x over previous
import functools

import jax
import jax.numpy as jnp
from jax import lax
from jax.experimental import pallas as pl
from jax.experimental.pallas import tpu as pltpu
from jax.experimental.pallas import tpu_sc as plsc

NC = 2
NS = 16
NW = NC * NS
LANES = 16
CHUNK = 8


def _sc_kernel(B, L, W_words, V_pad, HALF):
    rows_per_tile = B // NW
    n_pairs = rows_per_tile // (2 * CHUNK)
    n_full = L // LANES
    tail = L - n_full * LANES

    mesh = plsc.VectorSubcoreMesh(
        core_axis_name="c", subcore_axis_name="s",
        num_cores=NC, num_subcores=NS)

    @functools.partial(
        pl.kernel,
        out_type=jax.ShapeDtypeStruct((B,), jnp.float32),
        mesh=mesh,
        scratch_types=[
            pltpu.VMEM((W_words,), jnp.int32),
            pltpu.VMEM((V_pad,), jnp.float32),
            pltpu.VMEM((2, CHUNK, L), jnp.int32),
            pltpu.VMEM((rows_per_tile,), jnp.float32),
            pltpu.VMEM((LANES,), jnp.float32),
            pltpu.SemaphoreType.DMA,
            pltpu.SemaphoreType.DMA,
            pltpu.SemaphoreType.DMA,
        ],
        compiler_params=pltpu.CompilerParams(needs_layout_passes=False),
    )
    def body(g2l_hbm, emb_hbm, rules_hbm, bias_hbm, out_hbm,
             g2l_v, emb_v, rules_c, out_v, bias_v, sem, sem_a, sem_b):
        wid = lax.axis_index("s") * NC + lax.axis_index("c")
        row0 = wid * rows_per_tile

        c1 = pltpu.async_copy(g2l_hbm, g2l_v, sem)
        c2 = pltpu.async_copy(emb_hbm, emb_v, sem)
        c4 = pltpu.async_copy(bias_hbm, bias_v, sem)

        def fetch(rows_base, buf, s):
            return pltpu.async_copy(
                rules_hbm.at[pl.ds(rows_base, CHUNK), :], rules_c.at[buf], s)

        fetch(row0, 0, sem_a)
        fetch(row0 + CHUNK, 1, sem_b)

        c1.wait()
        c2.wait()
        c4.wait()

        lane = lax.iota(jnp.int32, LANES)
        m_tail = lane >= (LANES - tail)
        bias_vec = bias_v[...]

        def sq16(r):
            in_hi = r >= HALF
            word_idx = jnp.where(in_hi, r - HALF, r)
            w = plsc.load_gather(g2l_v, [word_idx])
            hi = jnp.bitwise_and(jnp.right_shift(w, 16), 0xFFFF)
            lo = jnp.bitwise_and(w, 0xFFFF)
            local = jnp.where(in_hi, hi, lo)
            v = plsc.load_gather(emb_v, [local])
            return v * v

        def chunk_sum(buf, base_lane, acc):
            ref = rules_c.at[buf]
            for r in range(CHUNK):
                s = jnp.zeros((LANES,), jnp.float32)
                for j in range(n_full):
                    s = s + sq16(ref[r, pl.ds(j * LANES, LANES)])
                if tail:
                    sqt = sq16(ref[r, pl.ds(L - LANES, LANES)])
                    s = s + jnp.where(m_tail, sqt, 0.0)
                acc = jnp.where(lane == base_lane + r, jnp.sum(s), acc)
            return acc

        def pair(i, carry):
            acc = jnp.zeros((LANES,), jnp.float32)
            base = row0 + i * (2 * CHUNK)
            pltpu.make_async_copy(
                rules_hbm.at[pl.ds(base, CHUNK), :], rules_c.at[0], sem_a
            ).wait()
            acc = chunk_sum(0, 0, acc)

            @pl.when(i < n_pairs - 1)
            def _():
                fetch(base + 2 * CHUNK, 0, sem_a)

            pltpu.make_async_copy(
                rules_hbm.at[pl.ds(base + CHUNK, CHUNK), :], rules_c.at[1], sem_b
            ).wait()
            acc = chunk_sum(1, CHUNK, acc)

            @pl.when(i < n_pairs - 1)
            def _():
                fetch(base + 3 * CHUNK, 1, sem_b)

            out_v[pl.ds(i * LANES, LANES)] = acc + bias_vec
            return carry

        lax.fori_loop(0, n_pairs, pair, 0)
        pltpu.sync_copy(out_v, out_hbm.at[pl.ds(row0, rows_per_tile)])

    return body


def kernel(rules, global_to_local, emb_weight, bias):
    B, L = rules.shape
    V = emb_weight.shape[0]
    G = global_to_local.shape[0]

    gp = global_to_local.astype(jnp.int32)
    half = (G + 1) // 2
    packed = jnp.bitwise_or(gp[:half], jnp.left_shift(gp[half:2 * half], 16))
    W_words = (half + 15) // 16 * 16
    packed = jnp.pad(packed, (0, W_words - half))

    V_pad = (V + 15) // 16 * 16
    emb_p = jnp.pad(emb_weight.reshape(-1), (0, V_pad - V))

    bias_vec = jnp.broadcast_to(bias.reshape(()), (LANES,)).astype(jnp.float32)
    rules_i32 = rules.astype(jnp.int32)

    out = _sc_kernel(B, L, W_words, V_pad, half)(packed, emb_p, rules_i32, bias_vec)
    return out.reshape(B, 1)
